# baseline (speedup 1.0000x reference)
import functools

import jax
import jax.numpy as jnp
from jax import lax
from jax.experimental import pallas as pl
from jax.experimental.pallas import tpu as pltpu
from jax.experimental.pallas import tpu_sc as plsc

NC = 2
NS = 16
NW = NC * NS
CH = 80


def _make_gather(N, E, H):
    per_w = E // NW
    n_ch = per_w // CH
    mesh = plsc.VectorSubcoreMesh(core_axis_name="c", subcore_axis_name="s")

    @functools.partial(
        pl.kernel,
        out_type=(
            jax.ShapeDtypeStruct((E, H), jnp.float32),
            jax.ShapeDtypeStruct((E, H), jnp.float32),
            jax.ShapeDtypeStruct((E,), jnp.float32),
        ),
        mesh=mesh,
        scratch_types=[
            [pltpu.VMEM((CH,), jnp.int32)] * 2,
            [pltpu.VMEM((CH,), jnp.int32)] * 2,
            [pltpu.VMEM((CH, H), jnp.float32)] * 2,
            [pltpu.VMEM((CH, H), jnp.float32)] * 2,
            [pltpu.VMEM((CH, H), jnp.float32)] * 2,
            [pltpu.VMEM((CH, H), jnp.float32)] * 2,
            [pltpu.VMEM((CH,), jnp.float32)] * 2,
            [pltpu.SemaphoreType.DMA] * 2,
            [pltpu.SemaphoreType.DMA] * 2,
            [pltpu.SemaphoreType.DMA] * 2,
        ],
    )
    def gather_k(h_hbm, c128_hbm, src_hbm, dst_hbm,
                 hs_out, hd_out, r2_out,
                 sidx, didx, hs_b, hd_b, cs_b, cd_b, r2_b,
                 sem_l, sem_g, sem_w):
        wid = lax.axis_index("s") * NC + lax.axis_index("c")
        base0 = wid * per_w

        def fire_l(b, j):
            base = base0 + j * CH
            pltpu.async_copy(src_hbm.at[pl.ds(base, CH)], sidx[b], sem_l[b])
            pltpu.async_copy(dst_hbm.at[pl.ds(base, CH)], didx[b], sem_l[b])

        def wait_l(b):
            pltpu.make_async_copy(src_hbm.at[pl.ds(0, CH)], sidx[b],
                                  sem_l[b]).wait()
            pltpu.make_async_copy(dst_hbm.at[pl.ds(0, CH)], didx[b],
                                  sem_l[b]).wait()

        def fire_g(b):
            pltpu.async_copy(h_hbm.at[sidx[b]], hs_b[b], sem_g[b])
            pltpu.async_copy(h_hbm.at[didx[b]], hd_b[b], sem_g[b])
            pltpu.async_copy(c128_hbm.at[sidx[b]], cs_b[b], sem_g[b])
            pltpu.async_copy(c128_hbm.at[didx[b]], cd_b[b], sem_g[b])

        def wait_g(b):
            pltpu.make_async_copy(h_hbm.at[sidx[b]], hs_b[b], sem_g[b]).wait()
            pltpu.make_async_copy(h_hbm.at[didx[b]], hd_b[b], sem_g[b]).wait()
            pltpu.make_async_copy(c128_hbm.at[sidx[b]], cs_b[b],
                                  sem_g[b]).wait()
            pltpu.make_async_copy(c128_hbm.at[didx[b]], cd_b[b],
                                  sem_g[b]).wait()

        lane = lax.iota(jnp.int32, 16)

        def r2comp(b):
            def body(g, c):
                z = jnp.zeros((16,), jnp.float32)
                for l in range(16):
                    i = g * 16 + l
                    v = (cs_b[b][i, pl.ds(0, 16)]
                         - cd_b[b][i, pl.ds(0, 16)])
                    d2 = v * v
                    z = jnp.where(lane == l, d2[0] + d2[1] + d2[2], z)
                r2_b[b][pl.ds(g * 16, 16)] = z
                return c
            lax.fori_loop(0, CH // 16, body, 0)

        def fire_w(b, j):
            base = base0 + j * CH
            pltpu.async_copy(hs_b[b], hs_out.at[pl.ds(base, CH)], sem_w[b])
            pltpu.async_copy(hd_b[b], hd_out.at[pl.ds(base, CH)], sem_w[b])
            pltpu.async_copy(r2_b[b], r2_out.at[pl.ds(base, CH)], sem_w[b])

        def wait_w(b):
            z = pl.ds(0, CH)
            pltpu.make_async_copy(hs_b[b], hs_out.at[z], sem_w[b]).wait()
            pltpu.make_async_copy(hd_b[b], hd_out.at[z], sem_w[b]).wait()
            pltpu.make_async_copy(r2_b[b], r2_out.at[z], sem_w[b]).wait()

        fire_l(0, 0)
        wait_l(0); fire_g(0)
        fire_l(1, 1)
        wait_g(0); fire_l(0, 2); r2comp(0); fire_w(0, 0)
        wait_l(1); fire_g(1)
        wait_g(1); fire_l(1, 3); r2comp(1); fire_w(1, 1)

        n_pairs = (n_ch - 2) // 2
        last = n_ch - 1

        def body(k, carry):
            for b in (0, 1):
                j = 2 * k + b
                wait_l(b)
                wait_w(b)
                fire_g(b)
                wait_g(b)
                jn = jnp.minimum(j + 2, last)
                fire_l(b, jn)
                r2comp(b)
                fire_w(b, j)
            return carry

        lax.fori_loop(1, 1 + n_pairs, body, 0)

        if n_ch % 2:
            wait_l(0)
            wait_w(0)
            fire_g(0)
            wait_g(0)
            r2comp(0)
            fire_w(0, last)
            wait_l(1)
            wait_w(1)
            wait_w(0)
        else:
            wait_l(0); wait_l(1)
            wait_w(0); wait_w(1)

    return gather_k


def _make_scatter(N, E, H):
    per_t = E // NS
    n_ch = per_t // CH
    rpt = (N // NS) // 8 * 8
    rem = N - NS * rpt
    mesh = plsc.VectorSubcoreMesh(core_axis_name="c", subcore_axis_name="s")

    @functools.partial(
        pl.kernel,
        out_type=(
            jax.ShapeDtypeStruct((N, H), jnp.float32),
            jax.ShapeDtypeStruct((N, H), jnp.float32),
        ),
        mesh=mesh,
        scratch_types=[
            [pltpu.VMEM((CH,), jnp.int32)] * 2,
            [pltpu.VMEM((CH,), jnp.int32)] * 2,
            [pltpu.VMEM((CH, H), jnp.float32)] * 2,
            [pltpu.VMEM((CH,), jnp.float32)] * 2,
            pltpu.VMEM_SHARED((N, H), jnp.float32),
            [pltpu.SemaphoreType.DMA] * 2,
            [pltpu.SemaphoreType.DMA] * 2,
            [pltpu.SemaphoreType.DMA] * 2,
        ],
    )
    def scatter_k(msgh_hbm, w_hbm, dst_hbm, src_hbm, c128s_hbm, zh_hbm,
                  hagg_out, xagg_out,
                  didx, sidx, m_b, w_b, acc, sem_l, sem_g, sem_a):
        cid = lax.axis_index("c")
        sid = lax.axis_index("s")
        base0 = sid * per_t
        r0 = sid * rpt
        pltpu.sync_copy(zh_hbm.at[pl.ds(r0, rpt)], acc.at[pl.ds(r0, rpt)])
        if rem:
            @pl.when(sid == NS - 1)
            def _():
                pltpu.sync_copy(zh_hbm.at[pl.ds(NS * rpt, rem)],
                                acc.at[pl.ds(NS * rpt, rem)])
        plsc.subcore_barrier()

        def wait_didx(b):
            pltpu.make_async_copy(dst_hbm.at[pl.ds(0, CH)], didx[b],
                                  sem_l[b]).wait()

        def fire_didx(b, j):
            base = base0 + j * CH
            pltpu.async_copy(dst_hbm.at[pl.ds(base, CH)], didx[b], sem_l[b])

        def fire_a(b):
            pltpu.async_copy(m_b[b], acc.at[didx[b]], sem_a[b], add=True)

        def wait_a(b):
            pltpu.make_async_copy(m_b[b], acc.at[didx[b]], sem_a[b]).wait()

        def run_h():
            def fire_l(b, j):
                base = base0 + j * CH
                fire_didx(b, j)
                pltpu.async_copy(msgh_hbm.at[pl.ds(base, CH)], m_b[b],
                                 sem_l[b])

            def wait_l(b):
                wait_didx(b)
                pltpu.make_async_copy(msgh_hbm.at[pl.ds(0, CH)], m_b[b],
                                      sem_l[b]).wait()

            fire_l(0, 0)
            fire_l(1, 1)
            last = n_ch - 1

            def body(k, carry):
                for b in (0, 1):
                    j = 2 * k + b
                    wait_l(b)
                    fire_a(b)
                    wait_a(b)
                    jn = jnp.minimum(j + 2, last)
                    fire_l(b, jn)
                return carry

            lax.fori_loop(0, n_ch // 2, body, 0)
            wait_l(0)
            wait_l(1)

        def run_x():
            def fire_lw(b, j):
                base = base0 + j * CH
                pltpu.async_copy(src_hbm.at[pl.ds(base, CH)], sidx[b],
                                 sem_l[b])
                pltpu.async_copy(w_hbm.at[pl.ds(base, CH)], w_b[b], sem_l[b])

            def wait_lw(b):
                pltpu.make_async_copy(src_hbm.at[pl.ds(0, CH)], sidx[b],
                                      sem_l[b]).wait()
                pltpu.make_async_copy(w_hbm.at[pl.ds(0, CH)], w_b[b],
                                      sem_l[b]).wait()

            def fire_g(b):
                pltpu.async_copy(c128s_hbm.at[sidx[b]], m_b[b], sem_g[b])

            def wait_g(b):
                pltpu.make_async_copy(c128s_hbm.at[sidx[b]], m_b[b],
                                      sem_g[b]).wait()

            def scale(b):
                def sbody(g, c):
                    wg = w_b[b][pl.ds(g * 16, 16)]
                    for l in range(16):
                        i = g * 16 + l
                        m_b[b][i, pl.ds(0, 16)] = (m_b[b][i, pl.ds(0, 16)]
                                                   * wg[l])
                    return c
                lax.fori_loop(0, CH // 16, sbody, 0)

            last = n_ch - 1

            def step(b, j, jn, first):
                wait_lw(b)
                if not first:
                    wait_a(b)
                fire_didx(b, j)
                fire_g(b)
                wait_g(b)
                scale(b)
                wait_didx(b)
                fire_a(b)
                fire_lw(b, jn)

            fire_lw(0, 0)
            fire_lw(1, 1)
            step(0, 0, 2, True)
            step(1, 1, 3, True)

            def body(k, carry):
                for b in (0, 1):
                    j = 2 * k + b
                    step(b, j, jnp.minimum(j + 2, last), False)
                return carry

            lax.fori_loop(1, n_ch // 2, body, 0)
            wait_a(0); wait_a(1)
            wait_lw(0); wait_lw(1)

        @pl.when(cid == 0)
        def _():
            run_h()

        @pl.when(cid == 1)
        def _():
            run_x()

        plsc.subcore_barrier()

        @pl.when(cid == 0)
        def _():
            pltpu.sync_copy(acc.at[pl.ds(r0, rpt)],
                            hagg_out.at[pl.ds(r0, rpt)])
            if rem:
                @pl.when(sid == NS - 1)
                def _():
                    pltpu.sync_copy(acc.at[pl.ds(NS * rpt, rem)],
                                    hagg_out.at[pl.ds(NS * rpt, rem)])

        @pl.when(cid == 1)
        def _():
            pltpu.sync_copy(acc.at[pl.ds(r0, rpt)],
                            xagg_out.at[pl.ds(r0, rpt)])
            if rem:
                @pl.when(sid == NS - 1)
                def _():
                    pltpu.sync_copy(acc.at[pl.ds(NS * rpt, rem)],
                                    xagg_out.at[pl.ds(NS * rpt, rem)])

    return scatter_k


def _silu2(xh):
    return xh * (jnp.tanh(xh) + 1.0)


def _edge_block_kernel(hs, hd, r2_ref, a_ref,
                       w1s, w1d, w1a, w1r, b1,
                       we1t, be1, watt, batt,
                       wc1t, bc1, wc2,
                       msgh_out, w_out):
    H = hs.shape[1]
    bf16 = jnp.bfloat16
    f32 = jnp.float32
    hs_ = hs[...].astype(bf16)
    hd_ = hd[...].astype(bf16)
    r2 = jnp.swapaxes(r2_ref[...].reshape(1, -1), 0, 1)
    r = jnp.sqrt(r2)
    pre = (jnp.dot(hs_, w1s[...], preferred_element_type=f32)
           + jnp.dot(hd_, w1d[...], preferred_element_type=f32)
           + jnp.dot(a_ref[...], w1a[...], preferred_element_type=f32)
           + r * w1r[...] + b1[...])
    m_e = _silu2(pre[:, :H]).astype(bf16)
    mh = _silu2(jnp.dot(m_e, we1t[...],
                        preferred_element_type=f32) + be1[...])
    att_t = jnp.tanh(
        jnp.sum(mh * watt[...], axis=1, keepdims=True) + batt[0, 0]) + 1.0
    msgh_out[...] = att_t * mh
    m1 = _silu2(pre[:, H:]).astype(bf16)
    m2 = _silu2(jnp.dot(m1, wc1t[...],
                        preferred_element_type=f32) + bc1[...])
    s = jnp.sum(m2 * wc2[...], axis=1, keepdims=True)
    w_out[...] = jnp.swapaxes(s / (r + 1.0), 0, 1).reshape(w_out.shape)


def _node_block_kernel(h, c128, hagg, xagg,
                       wn0h, wn0a, bn0, wn1t, bn1,
                       hout, cout):
    h_ = h[...]
    t = _silu2(jnp.dot(h_, wn0h[...], preferred_element_type=jnp.float32)
               + jnp.dot(hagg[...], wn0a[...],
                         preferred_element_type=jnp.float32)
               + bn0[...])
    hout[...] = h_ + jnp.dot(t, wn1t[...],
                             preferred_element_type=jnp.float32) + bn1[...]
    x = xagg[...]
    wsum = lax.slice(x, (0, 3), (x.shape[0], 4))
    cout[...] = c128[...] * (1.0 - wsum) + x


def kernel(h, coords, a, edge_index, W_e0, b_e0, W_e1, b_e1, W_att, b_att,
           W_n0, b_n0, W_n1, b_n1, W_c0, b_c0, W_c1, b_c1, W_c2):
    N, H = h.shape
    E = a.shape[0]
    DE = a.shape[1]
    f32 = jnp.float32

    bf16 = jnp.bfloat16
    src = edge_index[0]
    dst = edge_index[1]
    c128 = jnp.pad(coords, ((0, 0), (0, H - coords.shape[1])))
    c128s = c128.at[:, 3].set(1.0)

    hs, hd, r2 = _make_gather(N, E, H)(h, c128, src, dst)

    w1s = (0.5 * jnp.concatenate([W_e0[:, :H], W_c0[:, :H]],
                                 axis=0).T).astype(bf16)
    w1d = (0.5 * jnp.concatenate([W_e0[:, H:2 * H], W_c0[:, H:2 * H]],
                                 axis=0).T).astype(bf16)
    w1a = (0.5 * jnp.concatenate([W_e0[:, 2 * H + 1:], W_c0[:, 2 * H + 1:]],
                                 axis=0).T).astype(bf16)
    w1r = 0.5 * jnp.concatenate([W_e0[:, 2 * H],
                                 W_c0[:, 2 * H]])[None, :]
    b1 = 0.5 * jnp.concatenate([b_e0, b_c0])[None, :]
    we1t = (0.5 * W_e1.T).astype(bf16)
    be1 = 0.5 * b_e1[None, :]
    watt = 0.5 * W_att
    batt = 0.5 * b_att[None, :]
    wc1t = (0.5 * W_c1.T).astype(bf16)
    bc1 = 0.5 * b_c1[None, :]
    wc2 = W_c2
    a_bf = a.astype(bf16)

    BE = 4000
    n_eb = E // BE
    r2m = r2.reshape(n_eb, 1, BE)
    full = lambda shape: pl.BlockSpec(shape, lambda i: (0,) * len(shape))
    eb = lambda w: pl.BlockSpec((BE, w), lambda i: (i, 0))
    msgh, wmat = pl.pallas_call(
        _edge_block_kernel,
        grid=(n_eb,),
        in_specs=[
            eb(H), eb(H), pl.BlockSpec((1, 1, BE), lambda i: (i, 0, 0)),
            eb(DE),
            full((H, 2 * H)), full((H, 2 * H)), full((DE, 2 * H)),
            full((1, 2 * H)), full((1, 2 * H)),
            full((H, H)), full((1, H)), full((1, H)), full((1, 1)),
            full((H, H)), full((1, H)), full((1, H)),
        ],
        out_specs=[eb(H), pl.BlockSpec((1, 1, BE), lambda i: (i, 0, 0))],
        out_shape=[
            jax.ShapeDtypeStruct((E, H), f32),
            jax.ShapeDtypeStruct((n_eb, 1, BE), f32),
        ],
    )(hs, hd, r2m, a_bf, w1s, w1d, w1a, w1r, b1,
      we1t, be1, watt, batt, wc1t, bc1, wc2)
    w_flat = wmat.reshape(E)

    zh = jnp.zeros((N, H), f32)
    hagg, xagg = _make_scatter(N, E, H)(msgh, w_flat, dst, src, c128s, zh)

    wn0h = 0.5 * W_n0[:, :H].T
    wn0a = 0.25 * W_n0[:, H:].T
    bn0 = 0.5 * b_n0[None, :]
    wn1t = W_n1.T
    bn1 = b_n1[None, :]
    BN = 2000
    n_nb = N // BN
    nb = lambda w: pl.BlockSpec((BN, w), lambda i: (i, 0))
    hout, cout128 = pl.pallas_call(
        _node_block_kernel,
        grid=(n_nb,),
        in_specs=[
            nb(H), nb(H), nb(H), nb(H),
            full((H, H)), full((H, H)), full((1, H)),
            full((H, H)), full((1, H)),
        ],
        out_specs=[nb(H), nb(H)],
        out_shape=[
            jax.ShapeDtypeStruct((N, H), f32),
            jax.ShapeDtypeStruct((N, H), f32),
        ],
    )(h, c128, hagg, xagg, wn0h, wn0a, bn0, wn1t, bn1)

    return hout, cout128[:, :coords.shape[1]]


# speedup vs baseline: 1.1511x; 1.1511x over previous
import functools

import jax
import jax.numpy as jnp
from jax import lax
from jax.experimental import pallas as pl
from jax.experimental.pallas import tpu as pltpu
from jax.experimental.pallas import tpu_sc as plsc

NC = 2
NS = 16
NW = NC * NS
CH = 80


def _make_gather(N, H, e_off, e_len):
    per_w = e_len // NW
    n_ch = per_w // CH
    mesh = plsc.VectorSubcoreMesh(core_axis_name="c", subcore_axis_name="s")

    @functools.partial(
        pl.kernel,
        out_type=(
            jax.ShapeDtypeStruct((e_len, H), jnp.float32),
            jax.ShapeDtypeStruct((e_len, H), jnp.float32),
            jax.ShapeDtypeStruct((e_len, H), jnp.float32),
        ),
        mesh=mesh,
        scratch_types=[
            [pltpu.VMEM((CH,), jnp.int32)] * 2,
            [pltpu.VMEM((CH,), jnp.int32)] * 2,
            [pltpu.VMEM((CH, H), jnp.float32)] * 2,
            [pltpu.VMEM((CH, H), jnp.float32)] * 2,
            [pltpu.VMEM((CH, H), jnp.float32)] * 2,
            [pltpu.VMEM((CH, H), jnp.float32)] * 2,
            [pltpu.SemaphoreType.DMA] * 2,
            [pltpu.SemaphoreType.DMA] * 2,
            [pltpu.SemaphoreType.DMA] * 2,
        ],
    )
    def gather_k(h_hbm, c128_hbm, src_hbm, dst_hbm,
                 hs_out, hd_out, d_out,
                 sidx, didx, hs_b, hd_b, cs_b, cd_b,
                 sem_l, sem_g, sem_w):
        wid = lax.axis_index("s") * NC + lax.axis_index("c")
        base0 = wid * per_w

        def fire_l(b, j):
            base = e_off + base0 + j * CH
            pltpu.async_copy(src_hbm.at[pl.ds(base, CH)], sidx[b], sem_l[b])
            pltpu.async_copy(dst_hbm.at[pl.ds(base, CH)], didx[b], sem_l[b])

        def wait_l(b):
            pltpu.make_async_copy(src_hbm.at[pl.ds(0, CH)], sidx[b],
                                  sem_l[b]).wait()
            pltpu.make_async_copy(dst_hbm.at[pl.ds(0, CH)], didx[b],
                                  sem_l[b]).wait()

        def fire_g(b):
            pltpu.async_copy(h_hbm.at[sidx[b]], hs_b[b], sem_g[b])
            pltpu.async_copy(h_hbm.at[didx[b]], hd_b[b], sem_g[b])
            pltpu.async_copy(c128_hbm.at[sidx[b]], cs_b[b], sem_g[b])
            pltpu.async_copy(c128_hbm.at[didx[b]], cd_b[b], sem_g[b])

        def wait_g(b):
            pltpu.make_async_copy(h_hbm.at[sidx[b]], hs_b[b], sem_g[b]).wait()
            pltpu.make_async_copy(h_hbm.at[didx[b]], hd_b[b], sem_g[b]).wait()
            pltpu.make_async_copy(c128_hbm.at[sidx[b]], cs_b[b],
                                  sem_g[b]).wait()
            pltpu.make_async_copy(c128_hbm.at[didx[b]], cd_b[b],
                                  sem_g[b]).wait()

        def diffs(b):
            def sub_row(i, c):
                cs_b[b][i, pl.ds(0, 16)] = (cs_b[b][i, pl.ds(0, 16)]
                                            - cd_b[b][i, pl.ds(0, 16)])
                return c
            lax.fori_loop(0, CH, sub_row, 0)

        def fire_w(b, j):
            base = base0 + j * CH
            pltpu.async_copy(hs_b[b], hs_out.at[pl.ds(base, CH)], sem_w[b])
            pltpu.async_copy(hd_b[b], hd_out.at[pl.ds(base, CH)], sem_w[b])
            pltpu.async_copy(cs_b[b], d_out.at[pl.ds(base, CH)], sem_w[b])

        def wait_w(b):
            z = pl.ds(0, CH)
            pltpu.make_async_copy(hs_b[b], hs_out.at[z], sem_w[b]).wait()
            pltpu.make_async_copy(hd_b[b], hd_out.at[z], sem_w[b]).wait()
            pltpu.make_async_copy(cs_b[b], d_out.at[z], sem_w[b]).wait()

        fire_l(0, 0)
        wait_l(0); fire_g(0)
        fire_l(1, 1)
        wait_g(0); fire_l(0, 2); diffs(0); fire_w(0, 0)
        wait_l(1); fire_g(1)
        wait_g(1); fire_l(1, 3); diffs(1); fire_w(1, 1)

        n_pairs = (n_ch - 2) // 2
        last = n_ch - 1

        def body(k, carry):
            for b in (0, 1):
                j = 2 * k + b
                wait_l(b)
                wait_w(b)
                fire_g(b)
                wait_g(b)
                jn = jnp.minimum(j + 2, last)
                fire_l(b, jn)
                diffs(b)
                fire_w(b, j)
            return carry

        lax.fori_loop(1, 1 + n_pairs, body, 0)

        if n_ch % 2:
            wait_l(0)
            wait_w(0)
            fire_g(0)
            wait_g(0)
            diffs(0)
            fire_w(0, last)
            wait_l(1)
            wait_w(1)
            wait_w(0)
        else:
            wait_l(0); wait_l(1)
            wait_w(0); wait_w(1)

    return gather_k


def _make_scatter(N, H, len_a, len_b):
    rpt = (N // NS) // 8 * 8
    rem = N - NS * rpt
    mesh = plsc.VectorSubcoreMesh(core_axis_name="c", subcore_axis_name="s")

    @functools.partial(
        pl.kernel,
        out_type=(
            jax.ShapeDtypeStruct((N, H), jnp.float32),
            jax.ShapeDtypeStruct((N, H), jnp.float32),
        ),
        mesh=mesh,
        scratch_types=[
            [pltpu.VMEM((CH,), jnp.int32)] * 2,
            [pltpu.VMEM((CH, H), jnp.float32)] * 2,
            pltpu.VMEM_SHARED((N, H), jnp.float32),
            [pltpu.SemaphoreType.DMA] * 2,
            [pltpu.SemaphoreType.DMA] * 2,
        ],
    )
    def scatter_k(msgh_a, msgh_b, msgx_a, msgx_b, dst_hbm, zh_hbm,
                  hagg_out, xagg_out,
                  didx, m_b, acc, sem_l, sem_a):
        cid = lax.axis_index("c")
        sid = lax.axis_index("s")
        r0 = sid * rpt
        pltpu.sync_copy(zh_hbm.at[pl.ds(r0, rpt)], acc.at[pl.ds(r0, rpt)])
        if rem:
            @pl.when(sid == NS - 1)
            def _():
                pltpu.sync_copy(zh_hbm.at[pl.ds(NS * rpt, rem)],
                                acc.at[pl.ds(NS * rpt, rem)])
        plsc.subcore_barrier()

        def run_pipeline(src_ref, e_len, g_off):
            per_t = e_len // NS
            n_ch = per_t // CH
            base0 = sid * per_t

            def fire_l(b, j):
                base = base0 + j * CH
                pltpu.async_copy(dst_hbm.at[pl.ds(g_off + base, CH)],
                                 didx[b], sem_l[b])
                pltpu.async_copy(src_ref.at[pl.ds(base, CH)], m_b[b],
                                 sem_l[b])

            def wait_l(b):
                pltpu.make_async_copy(dst_hbm.at[pl.ds(0, CH)], didx[b],
                                      sem_l[b]).wait()
                pltpu.make_async_copy(src_ref.at[pl.ds(0, CH)], m_b[b],
                                      sem_l[b]).wait()

            def fire_a(b):
                pltpu.async_copy(m_b[b], acc.at[didx[b]], sem_a[b], add=True)

            def wait_a(b):
                pltpu.make_async_copy(m_b[b], acc.at[didx[b]],
                                      sem_a[b]).wait()

            fire_l(0, 0)
            fire_l(1, 1)
            last = n_ch - 1

            def body(k, carry):
                for b in (0, 1):
                    j = 2 * k + b
                    wait_l(b)
                    fire_a(b)
                    wait_a(b)
                    jn = jnp.minimum(j + 2, last)
                    fire_l(b, jn)
                return carry

            lax.fori_loop(0, n_ch // 2, body, 0)
            wait_l(0)
            wait_l(1)

        @pl.when(cid == 0)
        def _():
            run_pipeline(msgh_a, len_a, 0)
            run_pipeline(msgh_b, len_b, len_a)

        @pl.when(cid == 1)
        def _():
            run_pipeline(msgx_a, len_a, 0)
            run_pipeline(msgx_b, len_b, len_a)

        plsc.subcore_barrier()

        @pl.when(cid == 0)
        def _():
            pltpu.sync_copy(acc.at[pl.ds(r0, rpt)],
                            hagg_out.at[pl.ds(r0, rpt)])
            if rem:
                @pl.when(sid == NS - 1)
                def _():
                    pltpu.sync_copy(acc.at[pl.ds(NS * rpt, rem)],
                                    hagg_out.at[pl.ds(NS * rpt, rem)])

        @pl.when(cid == 1)
        def _():
            pltpu.sync_copy(acc.at[pl.ds(r0, rpt)],
                            xagg_out.at[pl.ds(r0, rpt)])
            if rem:
                @pl.when(sid == NS - 1)
                def _():
                    pltpu.sync_copy(acc.at[pl.ds(NS * rpt, rem)],
                                    xagg_out.at[pl.ds(NS * rpt, rem)])

    return scatter_k


def _silu2(xh):
    return xh * (jnp.tanh(xh) + 1.0)


def _edge_block_kernel(hs, hd, d_ref, a_ref,
                       w1s, w1d, w1a, w1r, b1,
                       we1t, be1, watt, batt,
                       wc1t, bc1, wc2,
                       msgh_out, msgx_out):
    H = hs.shape[1]
    bf16 = jnp.bfloat16
    f32 = jnp.float32
    hs_ = hs[...].astype(bf16)
    hd_ = hd[...].astype(bf16)
    d = d_ref[...]
    r2 = jnp.sum(d * d, axis=1, keepdims=True)
    r = jnp.sqrt(r2)
    pre = (jnp.dot(hs_, w1s[...], preferred_element_type=f32)
           + jnp.dot(hd_, w1d[...], preferred_element_type=f32)
           + jnp.dot(a_ref[...], w1a[...], preferred_element_type=f32)
           + r * w1r[...] + b1[...])
    m_e = _silu2(pre[:, :H]).astype(bf16)
    mh = _silu2(jnp.dot(m_e, we1t[...],
                        preferred_element_type=f32) + be1[...])
    att_t = jnp.tanh(
        jnp.sum(mh * watt[...], axis=1, keepdims=True) + batt[0, 0]) + 1.0
    msgh_out[...] = att_t * mh
    m1 = _silu2(pre[:, H:]).astype(bf16)
    m2 = _silu2(jnp.dot(m1, wc1t[...],
                        preferred_element_type=f32) + bc1[...])
    s = jnp.sum(m2 * wc2[...], axis=1, keepdims=True)
    msgx_out[...] = s * d / (r + 1.0)


def _node_block_kernel(h, c128, hagg, xagg,
                       wn0h, wn0a, bn0, wn1t, bn1,
                       hout, cout):
    h_ = h[...]
    t = _silu2(jnp.dot(h_, wn0h[...], preferred_element_type=jnp.float32)
               + jnp.dot(hagg[...], wn0a[...],
                         preferred_element_type=jnp.float32)
               + bn0[...])
    hout[...] = h_ + jnp.dot(t, wn1t[...],
                             preferred_element_type=jnp.float32) + bn1[...]
    cout[...] = c128[...] + xagg[...]


def kernel(h, coords, a, edge_index, W_e0, b_e0, W_e1, b_e1, W_att, b_att,
           W_n0, b_n0, W_n1, b_n1, W_c0, b_c0, W_c1, b_c1, W_c2):
    N, H = h.shape
    E = a.shape[0]
    DE = a.shape[1]
    f32 = jnp.float32

    bf16 = jnp.bfloat16
    src = edge_index[0]
    dst = edge_index[1]
    c128 = jnp.pad(coords, ((0, 0), (0, H - coords.shape[1])))

    GRAN = 12800
    EA = (E // 2) // GRAN * GRAN
    EB = E - EA

    w1s = (0.5 * jnp.concatenate([W_e0[:, :H], W_c0[:, :H]],
                                 axis=0).T).astype(bf16)
    w1d = (0.5 * jnp.concatenate([W_e0[:, H:2 * H], W_c0[:, H:2 * H]],
                                 axis=0).T).astype(bf16)
    w1a = (0.5 * jnp.concatenate([W_e0[:, 2 * H + 1:], W_c0[:, 2 * H + 1:]],
                                 axis=0).T).astype(bf16)
    w1r = 0.5 * jnp.concatenate([W_e0[:, 2 * H],
                                 W_c0[:, 2 * H]])[None, :]
    b1 = 0.5 * jnp.concatenate([b_e0, b_c0])[None, :]
    we1t = (0.5 * W_e1.T).astype(bf16)
    be1 = 0.5 * b_e1[None, :]
    watt = 0.5 * W_att
    batt = 0.5 * b_att[None, :]
    wc1t = (0.5 * W_c1.T).astype(bf16)
    bc1 = 0.5 * b_c1[None, :]
    wc2 = W_c2
    a_bf = a.astype(bf16)

    BE = 3200
    full = lambda shape: pl.BlockSpec(shape, lambda i: (0,) * len(shape))
    eb = lambda w: pl.BlockSpec((BE, w), lambda i: (i, 0))

    def edge_call(hs, hd, d, a_half, e_len):
        return pl.pallas_call(
            _edge_block_kernel,
            grid=(e_len // BE,),
            in_specs=[
                eb(H), eb(H), eb(H), eb(DE),
                full((H, 2 * H)), full((H, 2 * H)), full((DE, 2 * H)),
                full((1, 2 * H)), full((1, 2 * H)),
                full((H, H)), full((1, H)), full((1, H)), full((1, 1)),
                full((H, H)), full((1, H)), full((1, H)),
            ],
            out_specs=[eb(H), eb(H)],
            out_shape=[
                jax.ShapeDtypeStruct((e_len, H), f32),
                jax.ShapeDtypeStruct((e_len, H), f32),
            ],
        )(hs, hd, d, a_half, w1s, w1d, w1a, w1r, b1,
          we1t, be1, watt, batt, wc1t, bc1, wc2)

    hsA, hdA, dA = _make_gather(N, H, 0, EA)(h, c128, src, dst)
    hsB, hdB, dB = _make_gather(N, H, EA, EB)(h, c128, src, dst)
    msghA, msgxA = edge_call(hsA, hdA, dA, a_bf[:EA], EA)
    msghB, msgxB = edge_call(hsB, hdB, dB, a_bf[EA:], EB)

    zh = jnp.zeros((N, H), f32)
    hagg, xagg = _make_scatter(N, H, EA, EB)(
        msghA, msghB, msgxA, msgxB, dst, zh)

    wn0h = 0.5 * W_n0[:, :H].T
    wn0a = 0.25 * W_n0[:, H:].T
    bn0 = 0.5 * b_n0[None, :]
    wn1t = W_n1.T
    bn1 = b_n1[None, :]
    BN = 2000
    n_nb = N // BN
    nb = lambda w: pl.BlockSpec((BN, w), lambda i: (i, 0))
    hout, cout128 = pl.pallas_call(
        _node_block_kernel,
        grid=(n_nb,),
        in_specs=[
            nb(H), nb(H), nb(H), nb(H),
            full((H, H)), full((H, H)), full((1, H)),
            full((H, H)), full((1, H)),
        ],
        out_specs=[nb(H), nb(H)],
        out_shape=[
            jax.ShapeDtypeStruct((N, H), f32),
            jax.ShapeDtypeStruct((N, H), f32),
        ],
    )(h, c128, hagg, xagg, wn0h, wn0a, bn0, wn1t, bn1)

    return hout, cout128[:, :coords.shape[1]]


# speedup vs baseline: 1.2214x; 1.0611x over previous
import functools

import jax
import jax.numpy as jnp
from jax import lax
from jax.experimental import pallas as pl
from jax.experimental.pallas import tpu as pltpu
from jax.experimental.pallas import tpu_sc as plsc

NC = 2
NS = 16
NW = NC * NS
CH = 80


def _make_gather(N, H, e_off, e_len):
    per_w = e_len // NW
    n_ch = per_w // CH
    mesh = plsc.VectorSubcoreMesh(core_axis_name="c", subcore_axis_name="s")

    @functools.partial(
        pl.kernel,
        out_type=(
            jax.ShapeDtypeStruct((e_len, H), jnp.float32),
            jax.ShapeDtypeStruct((e_len, H), jnp.float32),
            jax.ShapeDtypeStruct((e_len, H), jnp.float32),
        ),
        mesh=mesh,
        scratch_types=[
            [pltpu.VMEM((CH,), jnp.int32)] * 2,
            [pltpu.VMEM((CH,), jnp.int32)] * 2,
            [pltpu.VMEM((CH, H), jnp.float32)] * 2,
            [pltpu.VMEM((CH, H), jnp.float32)] * 2,
            [pltpu.VMEM((CH, H), jnp.float32)] * 2,
            [pltpu.VMEM((CH, H), jnp.float32)] * 2,
            [pltpu.SemaphoreType.DMA] * 2,
            [pltpu.SemaphoreType.DMA] * 2,
            [pltpu.SemaphoreType.DMA] * 2,
        ],
    )
    def gather_k(h_hbm, c128_hbm, src_hbm, dst_hbm,
                 hs_out, hd_out, d_out,
                 sidx, didx, hs_b, hd_b, cs_b, cd_b,
                 sem_l, sem_g, sem_w):
        wid = lax.axis_index("s") * NC + lax.axis_index("c")
        base0 = wid * per_w

        def fire_l(b, j):
            base = e_off + base0 + j * CH
            pltpu.async_copy(src_hbm.at[pl.ds(base, CH)], sidx[b], sem_l[b])
            pltpu.async_copy(dst_hbm.at[pl.ds(base, CH)], didx[b], sem_l[b])

        def wait_l(b):
            pltpu.make_async_copy(src_hbm.at[pl.ds(0, CH)], sidx[b],
                                  sem_l[b]).wait()
            pltpu.make_async_copy(dst_hbm.at[pl.ds(0, CH)], didx[b],
                                  sem_l[b]).wait()

        def fire_g(b):
            pltpu.async_copy(h_hbm.at[sidx[b]], hs_b[b], sem_g[b])
            pltpu.async_copy(h_hbm.at[didx[b]], hd_b[b], sem_g[b])
            pltpu.async_copy(c128_hbm.at[sidx[b]], cs_b[b], sem_g[b])
            pltpu.async_copy(c128_hbm.at[didx[b]], cd_b[b], sem_g[b])

        def wait_g(b):
            pltpu.make_async_copy(h_hbm.at[sidx[b]], hs_b[b], sem_g[b]).wait()
            pltpu.make_async_copy(h_hbm.at[didx[b]], hd_b[b], sem_g[b]).wait()
            pltpu.make_async_copy(c128_hbm.at[sidx[b]], cs_b[b],
                                  sem_g[b]).wait()
            pltpu.make_async_copy(c128_hbm.at[didx[b]], cd_b[b],
                                  sem_g[b]).wait()

        def diffs(b):
            def sub_row(i, c):
                cs_b[b][i, pl.ds(0, 16)] = (cs_b[b][i, pl.ds(0, 16)]
                                            - cd_b[b][i, pl.ds(0, 16)])
                return c
            lax.fori_loop(0, CH, sub_row, 0)

        def fire_w(b, j):
            base = base0 + j * CH
            pltpu.async_copy(hs_b[b], hs_out.at[pl.ds(base, CH)], sem_w[b])
            pltpu.async_copy(hd_b[b], hd_out.at[pl.ds(base, CH)], sem_w[b])
            pltpu.async_copy(cs_b[b], d_out.at[pl.ds(base, CH)], sem_w[b])

        def wait_w(b):
            z = pl.ds(0, CH)
            pltpu.make_async_copy(hs_b[b], hs_out.at[z], sem_w[b]).wait()
            pltpu.make_async_copy(hd_b[b], hd_out.at[z], sem_w[b]).wait()
            pltpu.make_async_copy(cs_b[b], d_out.at[z], sem_w[b]).wait()

        fire_l(0, 0)
        wait_l(0); fire_g(0)
        fire_l(1, 1)
        wait_g(0); fire_l(0, 2); diffs(0); fire_w(0, 0)
        wait_l(1); fire_g(1)
        wait_g(1); fire_l(1, 3); diffs(1); fire_w(1, 1)

        n_pairs = (n_ch - 2) // 2
        last = n_ch - 1

        def body(k, carry):
            for b in (0, 1):
                j = 2 * k + b
                wait_l(b)
                wait_w(b)
                fire_g(b)
                wait_g(b)
                jn = jnp.minimum(j + 2, last)
                fire_l(b, jn)
                diffs(b)
                fire_w(b, j)
            return carry

        lax.fori_loop(1, 1 + n_pairs, body, 0)

        if n_ch % 2:
            wait_l(0)
            wait_w(0)
            fire_g(0)
            wait_g(0)
            diffs(0)
            fire_w(0, last)
            wait_l(1)
            wait_w(1)
            wait_w(0)
        else:
            wait_l(0); wait_l(1)
            wait_w(0); wait_w(1)

    return gather_k


def _make_scatter(N, H, e_off, e_len):
    per_t = e_len // NS
    n_ch = per_t // CH
    rpt = (N // NS) // 8 * 8
    rem = N - NS * rpt
    mesh = plsc.VectorSubcoreMesh(core_axis_name="c", subcore_axis_name="s")

    @functools.partial(
        pl.kernel,
        out_type=(
            jax.ShapeDtypeStruct((N, H), jnp.float32),
            jax.ShapeDtypeStruct((N, H), jnp.float32),
        ),
        mesh=mesh,
        scratch_types=[
            [pltpu.VMEM((CH,), jnp.int32)] * 2,
            [pltpu.VMEM((CH, H), jnp.float32)] * 2,
            pltpu.VMEM_SHARED((N, H), jnp.float32),
            [pltpu.SemaphoreType.DMA] * 2,
            [pltpu.SemaphoreType.DMA] * 2,
        ],
    )
    def scatter_k(msgh_hbm, msgx_hbm, dst_hbm, zh_hbm,
                  hagg_out, xagg_out,
                  didx, m_b, acc, sem_l, sem_a):
        cid = lax.axis_index("c")
        sid = lax.axis_index("s")
        r0 = sid * rpt
        pltpu.sync_copy(zh_hbm.at[pl.ds(r0, rpt)], acc.at[pl.ds(r0, rpt)])
        if rem:
            @pl.when(sid == NS - 1)
            def _():
                pltpu.sync_copy(zh_hbm.at[pl.ds(NS * rpt, rem)],
                                acc.at[pl.ds(NS * rpt, rem)])
        plsc.subcore_barrier()

        def run_pipeline(src_ref):
            base0 = sid * per_t

            def fire_l(b, j):
                base = base0 + j * CH
                pltpu.async_copy(dst_hbm.at[pl.ds(e_off + base, CH)],
                                 didx[b], sem_l[b])
                pltpu.async_copy(src_ref.at[pl.ds(base, CH)], m_b[b],
                                 sem_l[b])

            def wait_l(b):
                pltpu.make_async_copy(dst_hbm.at[pl.ds(0, CH)], didx[b],
                                      sem_l[b]).wait()
                pltpu.make_async_copy(src_ref.at[pl.ds(0, CH)], m_b[b],
                                      sem_l[b]).wait()

            def fire_a(b):
                pltpu.async_copy(m_b[b], acc.at[didx[b]], sem_a[b], add=True)

            def wait_a(b):
                pltpu.make_async_copy(m_b[b], acc.at[didx[b]],
                                      sem_a[b]).wait()

            fire_l(0, 0)
            fire_l(1, 1)
            last = n_ch - 1

            def body(k, carry):
                for b in (0, 1):
                    j = 2 * k + b
                    wait_l(b)
                    fire_a(b)
                    wait_a(b)
                    jn = jnp.minimum(j + 2, last)
                    fire_l(b, jn)
                return carry

            lax.fori_loop(0, n_ch // 2, body, 0)
            wait_l(0)
            wait_l(1)

        @pl.when(cid == 0)
        def _():
            run_pipeline(msgh_hbm)

        @pl.when(cid == 1)
        def _():
            run_pipeline(msgx_hbm)

        plsc.subcore_barrier()

        @pl.when(cid == 0)
        def _():
            pltpu.sync_copy(acc.at[pl.ds(r0, rpt)],
                            hagg_out.at[pl.ds(r0, rpt)])
            if rem:
                @pl.when(sid == NS - 1)
                def _():
                    pltpu.sync_copy(acc.at[pl.ds(NS * rpt, rem)],
                                    hagg_out.at[pl.ds(NS * rpt, rem)])

        @pl.when(cid == 1)
        def _():
            pltpu.sync_copy(acc.at[pl.ds(r0, rpt)],
                            xagg_out.at[pl.ds(r0, rpt)])
            if rem:
                @pl.when(sid == NS - 1)
                def _():
                    pltpu.sync_copy(acc.at[pl.ds(NS * rpt, rem)],
                                    xagg_out.at[pl.ds(NS * rpt, rem)])

    return scatter_k


def _silu2(xh):
    return xh * (jnp.tanh(xh) + 1.0)


def _edge_block_kernel(hs, hd, d_ref, a_ref,
                       w1s, w1d, w1a, w1r, b1,
                       we1t, be1, watt, batt,
                       wc1t, bc1, wc2,
                       msgh_out, msgx_out):
    H = hs.shape[1]
    bf16 = jnp.bfloat16
    f32 = jnp.float32
    hs_ = hs[...].astype(bf16)
    hd_ = hd[...].astype(bf16)
    d = d_ref[...]
    r2 = jnp.sum(d * d, axis=1, keepdims=True)
    r = jnp.sqrt(r2)
    pre = (jnp.dot(hs_, w1s[...], preferred_element_type=f32)
           + jnp.dot(hd_, w1d[...], preferred_element_type=f32)
           + jnp.dot(a_ref[...], w1a[...], preferred_element_type=f32)
           + r * w1r[...] + b1[...])
    m_e = _silu2(pre[:, :H]).astype(bf16)
    mh = _silu2(jnp.dot(m_e, we1t[...],
                        preferred_element_type=f32) + be1[...])
    att_t = jnp.tanh(
        jnp.sum(mh * watt[...], axis=1, keepdims=True) + batt[0, 0]) + 1.0
    msgh_out[...] = att_t * mh
    m1 = _silu2(pre[:, H:]).astype(bf16)
    m2 = _silu2(jnp.dot(m1, wc1t[...],
                        preferred_element_type=f32) + bc1[...])
    s = jnp.sum(m2 * wc2[...], axis=1, keepdims=True)
    msgx_out[...] = s * d / (r + 1.0)


def _node_block_kernel(h, c128, hagg_a, hagg_b, xagg_a, xagg_b,
                       wn0h, wn0a, bn0, wn1t, bn1,
                       hout, cout):
    h_ = h[...]
    hagg = hagg_a[...] + hagg_b[...]
    t = _silu2(jnp.dot(h_, wn0h[...], preferred_element_type=jnp.float32)
               + jnp.dot(hagg, wn0a[...],
                         preferred_element_type=jnp.float32)
               + bn0[...])
    hout[...] = h_ + jnp.dot(t, wn1t[...],
                             preferred_element_type=jnp.float32) + bn1[...]
    cout[...] = c128[...] + xagg_a[...] + xagg_b[...]


def kernel(h, coords, a, edge_index, W_e0, b_e0, W_e1, b_e1, W_att, b_att,
           W_n0, b_n0, W_n1, b_n1, W_c0, b_c0, W_c1, b_c1, W_c2):
    N, H = h.shape
    E = a.shape[0]
    DE = a.shape[1]
    f32 = jnp.float32

    bf16 = jnp.bfloat16
    src = edge_index[0]
    dst = edge_index[1]
    c128 = jnp.pad(coords, ((0, 0), (0, H - coords.shape[1])))

    GRAN = 12800
    EA = (E // 2) // GRAN * GRAN
    EB = E - EA

    w1s = (0.5 * jnp.concatenate([W_e0[:, :H], W_c0[:, :H]],
                                 axis=0).T).astype(bf16)
    w1d = (0.5 * jnp.concatenate([W_e0[:, H:2 * H], W_c0[:, H:2 * H]],
                                 axis=0).T).astype(bf16)
    w1a = (0.5 * jnp.concatenate([W_e0[:, 2 * H + 1:], W_c0[:, 2 * H + 1:]],
                                 axis=0).T).astype(bf16)
    w1r = 0.5 * jnp.concatenate([W_e0[:, 2 * H],
                                 W_c0[:, 2 * H]])[None, :]
    b1 = 0.5 * jnp.concatenate([b_e0, b_c0])[None, :]
    we1t = (0.5 * W_e1.T).astype(bf16)
    be1 = 0.5 * b_e1[None, :]
    watt = 0.5 * W_att
    batt = 0.5 * b_att[None, :]
    wc1t = (0.5 * W_c1.T).astype(bf16)
    bc1 = 0.5 * b_c1[None, :]
    wc2 = W_c2
    a_bf = a.astype(bf16)

    BE = 3200
    full = lambda shape: pl.BlockSpec(shape, lambda i: (0,) * len(shape))
    eb = lambda w: pl.BlockSpec((BE, w), lambda i: (i, 0))

    def edge_call(hs, hd, d, a_half, e_len):
        return pl.pallas_call(
            _edge_block_kernel,
            grid=(e_len // BE,),
            in_specs=[
                eb(H), eb(H), eb(H), eb(DE),
                full((H, 2 * H)), full((H, 2 * H)), full((DE, 2 * H)),
                full((1, 2 * H)), full((1, 2 * H)),
                full((H, H)), full((1, H)), full((1, H)), full((1, 1)),
                full((H, H)), full((1, H)), full((1, H)),
            ],
            out_specs=[eb(H), eb(H)],
            out_shape=[
                jax.ShapeDtypeStruct((e_len, H), f32),
                jax.ShapeDtypeStruct((e_len, H), f32),
            ],
        )(hs, hd, d, a_half, w1s, w1d, w1a, w1r, b1,
          we1t, be1, watt, batt, wc1t, bc1, wc2)

    hsA, hdA, dA = _make_gather(N, H, 0, EA)(h, c128, src, dst)
    hsB, hdB, dB = _make_gather(N, H, EA, EB)(h, c128, src, dst)
    msghA, msgxA = edge_call(hsA, hdA, dA, a_bf[:EA], EA)
    msghB, msgxB = edge_call(hsB, hdB, dB, a_bf[EA:], EB)

    zh = jnp.zeros((N, H), f32)
    haggA, xaggA = _make_scatter(N, H, 0, EA)(msghA, msgxA, dst, zh)
    haggB, xaggB = _make_scatter(N, H, EA, EB)(msghB, msgxB, dst, zh)

    wn0h = 0.5 * W_n0[:, :H].T
    wn0a = 0.25 * W_n0[:, H:].T
    bn0 = 0.5 * b_n0[None, :]
    wn1t = W_n1.T
    bn1 = b_n1[None, :]
    BN = 2000
    n_nb = N // BN
    nb = lambda w: pl.BlockSpec((BN, w), lambda i: (i, 0))
    hout, cout128 = pl.pallas_call(
        _node_block_kernel,
        grid=(n_nb,),
        in_specs=[
            nb(H), nb(H), nb(H), nb(H), nb(H), nb(H),
            full((H, H)), full((H, H)), full((1, H)),
            full((H, H)), full((1, H)),
        ],
        out_specs=[nb(H), nb(H)],
        out_shape=[
            jax.ShapeDtypeStruct((N, H), f32),
            jax.ShapeDtypeStruct((N, H), f32),
        ],
    )(h, c128, haggA, haggB, xaggA, xaggB, wn0h, wn0a, bn0, wn1t, bn1)

    return hout, cout128[:, :coords.shape[1]]
